# baseline (speedup 1.0000x reference)
import functools

import jax
import jax.numpy as jnp
from jax.experimental import pallas as pl
from jax.experimental.pallas import tpu as pltpu

_CHUNK = 1024
_NBUF = 6


def _dispatch_combine_pipelined(x_hbm, scores_hbm, out_hbm,
                                in_buf, sc_buf, out_buf,
                                in_sems, sc_sems, out_sems):
    num_tokens = x_hbm.shape[0]
    nchunks = num_tokens // _CHUNK

    def in_copy(i, slot):
        return pltpu.make_async_copy(
            x_hbm.at[pl.ds(i * _CHUNK, _CHUNK), :],
            in_buf.at[slot], in_sems.at[slot])

    def sc_copy(i, slot):
        return pltpu.make_async_copy(
            scores_hbm.at[pl.ds(i * _CHUNK, _CHUNK), :],
            sc_buf.at[slot], sc_sems.at[slot])

    def out_copy(i, slot):
        return pltpu.make_async_copy(
            out_buf.at[slot],
            out_hbm.at[pl.ds(i * _CHUNK, _CHUNK), :], out_sems.at[slot])

    for slot in range(min(_NBUF, nchunks)):
        in_copy(slot, slot).start()
        sc_copy(slot, slot).start()

    for i in range(nchunks):
        slot = i % _NBUF
        in_copy(i, slot).wait()
        sc_copy(i, slot).wait()
        if i >= _NBUF:
            out_copy(i - _NBUF, slot).wait()
        s = jnp.sum(sc_buf[slot], axis=1, keepdims=True)
        out_buf[slot] = in_buf[slot] * s
        out_copy(i, slot).start()
        j = i + _NBUF
        if j < nchunks:
            in_copy(j, slot).start()
            sc_copy(j, slot).start()

    for i in range(max(0, nchunks - _NBUF), nchunks):
        out_copy(i, i % _NBUF).wait()


@functools.partial(jax.jit, static_argnames=())
def _run(x, top_scores):
    num_tokens, dim = x.shape
    top_k = top_scores.shape[1]
    return pl.pallas_call(
        _dispatch_combine_pipelined,
        in_specs=[
            pl.BlockSpec(memory_space=pltpu.HBM),
            pl.BlockSpec(memory_space=pltpu.HBM),
        ],
        out_specs=pl.BlockSpec(memory_space=pltpu.HBM),
        out_shape=jax.ShapeDtypeStruct((num_tokens, dim), x.dtype),
        scratch_shapes=[
            pltpu.VMEM((_NBUF, _CHUNK, dim), x.dtype),
            pltpu.VMEM((_NBUF, _CHUNK, top_k), top_scores.dtype),
            pltpu.VMEM((_NBUF, _CHUNK, dim), x.dtype),
            pltpu.SemaphoreType.DMA((_NBUF,)),
            pltpu.SemaphoreType.DMA((_NBUF,)),
            pltpu.SemaphoreType.DMA((_NBUF,)),
        ],
    )(x, top_scores)


def kernel(x, top_scores, selected_experts_indices, num_tokens_per_expert):
    del selected_experts_indices, num_tokens_per_expert
    return _run(x, top_scores)
